# Initial kernel scaffold; baseline (speedup 1.0000x reference)
import jax
import jax.numpy as jnp
from jax.experimental import pallas as pl


def kernel(x, edge_index, W_enc, b_enc, W_conv, b_conv, W_res, b_res, W_dec, b_dec, weight_mlp, lamda1):
    raise NotImplementedError("write your pallas kernel here")


# speedup vs baseline: 7.5168x; 7.5168x over previous
import functools

import jax
import jax.numpy as jnp
from jax import lax
from jax.experimental import pallas as pl
from jax.experimental.pallas import tpu as pltpu
from jax.experimental.pallas import tpu_sc as plsc

N = 10000
E = 320000
H = 128
NCLASS = 40

_NTILE = 16
_NW = 32
_EPW = E // _NW
_K = 80
_NCH = _EPW // _K
_NP = 10240
_RPT = _NP // _NTILE
_ZR = 128
_ND = 10240
_DPT = _ND // _NTILE

_mesh = plsc.VectorSubcoreMesh(core_axis_name="c", subcore_axis_name="s")


@functools.partial(
    pl.kernel,
    mesh=_mesh,
    out_type=jax.ShapeDtypeStruct((2, _ND), jnp.float32),
    scratch_types=[
        pltpu.VMEM((_K,), jnp.int32),
        pltpu.VMEM((_K,), jnp.float32),
        pltpu.VMEM((_DPT,), jnp.float32),
        pltpu.VMEM_SHARED((_ND,), jnp.float32),
    ],
)
def _sc_degree(col_hbm, out_hbm, idxc, ones, zbuf, acc):
    c = lax.axis_index("c")
    s = lax.axis_index("s")
    wid = c * _NTILE + s

    def fill(i, _):
        ones[pl.ds(i * 16, 16)] = jnp.full((16,), 1.0, jnp.float32)
        zbuf[pl.ds(i * 16, 16)] = jnp.zeros((16,), jnp.float32)
        return 0
    lax.fori_loop(0, _K // 16, fill, 0)

    def zfill(i, _):
        zbuf[pl.ds(i * 16, 16)] = jnp.zeros((16,), jnp.float32)
        return 0
    lax.fori_loop(_K // 16, _DPT // 16, zfill, 0)

    pltpu.sync_copy(zbuf, acc.at[pl.ds(s * _DPT, _DPT)])
    plsc.subcore_barrier()

    base = wid * _EPW

    def chunk(k, _):
        pltpu.sync_copy(col_hbm.at[pl.ds(base + k * _K, _K)], idxc)
        pltpu.sync_copy(ones, acc.at[idxc], add=True)
        return 0
    lax.fori_loop(0, _NCH, chunk, 0)

    plsc.subcore_barrier()
    pltpu.sync_copy(acc.at[pl.ds(s * _DPT, _DPT)],
                    out_hbm.at[c, pl.ds(s * _DPT, _DPT)])


@functools.partial(
    pl.kernel,
    mesh=_mesh,
    out_type=jax.ShapeDtypeStruct((2, _NP, H), jnp.float32),
    scratch_types=[
        pltpu.VMEM((_K,), jnp.int32),
        pltpu.VMEM((_K,), jnp.int32),
        pltpu.VMEM((_K, H), jnp.float32),
        pltpu.VMEM((_K, H), jnp.float32),
        pltpu.VMEM_SHARED((_NP, H), jnp.float32),
        pltpu.SemaphoreType.DMA,
        pltpu.SemaphoreType.DMA,
    ],
)
def _sc_edge_s(t_hbm, row_hbm, col_hbm, out_hbm, idxr, idxc, a, b, acc, sem1, sem2):
    c = lax.axis_index("c")
    s = lax.axis_index("s")
    wid = c * _NTILE + s

    def zrow(r, _):
        for j in range(8):
            b[r, pl.ds(j * 16, 16)] = jnp.zeros((16,), jnp.float32)
        return 0
    lax.fori_loop(0, _ZR, zrow, 0)
    for t in range(_RPT // _ZR):
        pltpu.sync_copy(b.at[pl.ds(0, _ZR)],
                        acc.at[pl.ds(s * _RPT + t * _ZR, _ZR)])
    plsc.subcore_barrier()

    base = wid * _EPW

    def chunk(k, _):
        off = base + k * _K
        pltpu.sync_copy(row_hbm.at[pl.ds(off, _K)], idxr)
        pltpu.sync_copy(col_hbm.at[pl.ds(off, _K)], idxc)
        ga = pltpu.async_copy(t_hbm.at[idxr], a, sem1)
        gb = pltpu.async_copy(t_hbm.at[idxc], b, sem2)
        ga.wait()
        gb.wait()

        def relu_row(e, _):
            for j in range(8):
                sl = pl.ds(j * 16, 16)
                a[e, sl] = jnp.maximum(a[e, sl] - b[e, sl], 0.0)
            return 0
        lax.fori_loop(0, _K, relu_row, 0)
        pltpu.sync_copy(a, acc.at[idxc], add=True)
        return 0
    lax.fori_loop(0, _NCH, chunk, 0)

    plsc.subcore_barrier()
    pltpu.sync_copy(acc.at[pl.ds(s * _RPT, _RPT)],
                    out_hbm.at[c, pl.ds(s * _RPT, _RPT)])


@functools.partial(
    pl.kernel,
    mesh=_mesh,
    out_type=jax.ShapeDtypeStruct((2, _NP, H), jnp.float32),
    scratch_types=[
        pltpu.VMEM((_K,), jnp.int32),
        pltpu.VMEM((_K,), jnp.int32),
        pltpu.VMEM((_K, H), jnp.float32),
        pltpu.VMEM_SHARED((_NP, H), jnp.float32),
        pltpu.SemaphoreType.DMA,
    ],
)
def _sc_edge_c(t_hbm, row_hbm, col_hbm, out_hbm, idxr, idxc, a, acc, sem1):
    c = lax.axis_index("c")
    s = lax.axis_index("s")
    wid = c * _NTILE + s

    def zrow(r, _):
        for j in range(8):
            a[r, pl.ds(j * 16, 16)] = jnp.zeros((16,), jnp.float32)
        return 0
    lax.fori_loop(0, _ZR, zrow, 0)
    for t in range(_RPT // _ZR):
        pltpu.sync_copy(a.at[pl.ds(0, _ZR)],
                        acc.at[pl.ds(s * _RPT + t * _ZR, _ZR)])
    plsc.subcore_barrier()

    base = wid * _EPW

    def chunk(k, _):
        off = base + k * _K
        pltpu.sync_copy(row_hbm.at[pl.ds(off, _K)], idxr)
        pltpu.sync_copy(col_hbm.at[pl.ds(off, _K)], idxc)
        pltpu.async_copy(t_hbm.at[idxr], a, sem1).wait()
        pltpu.sync_copy(a, acc.at[idxc], add=True)
        return 0
    lax.fori_loop(0, _NCH, chunk, 0)

    plsc.subcore_barrier()
    pltpu.sync_copy(acc.at[pl.ds(s * _RPT, _RPT)],
                    out_hbm.at[c, pl.ds(s * _RPT, _RPT)])


_BR = 1000
_GRID = N // _BR


def _full(shape):
    return pl.BlockSpec(shape, lambda i: tuple(0 for _ in shape))


def _enc_body(x_ref, w_ref, b_ref, o_ref):
    o_ref[...] = jnp.maximum(
        jnp.dot(x_ref[...], w_ref[...], preferred_element_type=jnp.float32)
        + b_ref[...], 0.0)


def _enc(x, wt, b):
    return pl.pallas_call(
        _enc_body,
        grid=(_GRID,),
        in_specs=[pl.BlockSpec((_BR, H), lambda i: (i, 0)),
                  _full((H, H)), _full((1, H))],
        out_specs=pl.BlockSpec((_BR, H), lambda i: (i, 0)),
        out_shape=jax.ShapeDtypeStruct((N, H), jnp.float32),
    )(x, wt, b)


def _pre_body(x_ref, wmc_ref, wrt_ref, bv_ref, dinv_ref,
              xwm_ref, xwcs_ref, v_ref):
    z = jnp.dot(x_ref[...], wmc_ref[...], preferred_element_type=jnp.float32)
    xwm = z[:, :H]
    xwc = z[:, H:]
    xwm_ref[...] = xwm
    xwcs_ref[...] = xwc * dinv_ref[...]
    v_ref[...] = bv_ref[...] - jnp.dot(
        xwc, wrt_ref[...], preferred_element_type=jnp.float32)


def _pre(x, wmc, wrt, bv, dinv):
    return pl.pallas_call(
        _pre_body,
        grid=(_GRID,),
        in_specs=[pl.BlockSpec((_BR, H), lambda i: (i, 0)),
                  _full((H, 2 * H)), _full((H, H)), _full((1, H)),
                  pl.BlockSpec((_BR, 1), lambda i: (i, 0))],
        out_specs=[pl.BlockSpec((_BR, H), lambda i: (i, 0))] * 3,
        out_shape=[jax.ShapeDtypeStruct((N, H), jnp.float32)] * 3,
    )(x, wmc, wrt, bv, dinv)


def _post_body(x_ref, sp_ref, cp_ref, xwcs_ref, v_ref, dinv_ref, lam_ref, o_ref):
    ssum = sp_ref[0] + sp_ref[1]
    csum = cp_ref[0] + cp_ref[1]
    t = jnp.maximum(dinv_ref[...] * (csum + xwcs_ref[...]) + v_ref[...], 0.0)
    o_ref[...] = t + lam_ref[0, 0] * x_ref[...] * ssum


def _post(x, sp, cp, xwcs, v, dinv, lam):
    return pl.pallas_call(
        _post_body,
        grid=(_GRID,),
        in_specs=[pl.BlockSpec((_BR, H), lambda i: (i, 0)),
                  pl.BlockSpec((2, _BR, H), lambda i: (0, i, 0)),
                  pl.BlockSpec((2, _BR, H), lambda i: (0, i, 0)),
                  pl.BlockSpec((_BR, H), lambda i: (i, 0)),
                  pl.BlockSpec((_BR, H), lambda i: (i, 0)),
                  pl.BlockSpec((_BR, 1), lambda i: (i, 0)),
                  _full((1, 1))],
        out_specs=pl.BlockSpec((_BR, H), lambda i: (i, 0)),
        out_shape=jax.ShapeDtypeStruct((N, H), jnp.float32),
    )(x, sp, cp, xwcs, v, dinv, lam)


def _dec_body(x_ref, w_ref, b_ref, o_ref):
    o_ref[...] = jnp.dot(
        x_ref[...], w_ref[...], preferred_element_type=jnp.float32) + b_ref[...]


def _dec(x, wt, b):
    return pl.pallas_call(
        _dec_body,
        grid=(_GRID,),
        in_specs=[pl.BlockSpec((_BR, H), lambda i: (i, 0)),
                  _full((H, NCLASS)), _full((1, NCLASS))],
        out_specs=pl.BlockSpec((_BR, NCLASS), lambda i: (i, 0)),
        out_shape=jax.ShapeDtypeStruct((N, NCLASS), jnp.float32),
    )(x, wt, b)


def kernel(x, edge_index, W_enc, b_enc, W_conv, b_conv, W_res, b_res,
           W_dec, b_dec, weight_mlp, lamda1):
    row = edge_index[0]
    col = edge_index[1]

    degp = _sc_degree(col)
    deg = degp[0, :N] + degp[1, :N] + 1.0
    dinv = jax.lax.rsqrt(deg).reshape(N, 1)

    X = _enc(x, W_enc.T, b_enc.reshape(1, H))
    wmc = jnp.concatenate([weight_mlp, W_conv.T], axis=1)
    bv = (b_conv - b_res).reshape(1, H)
    lam = lamda1.reshape(1, 1)
    for _ in range(2):
        xwm, xwcs, v = _pre(X, wmc, W_res.T, bv, dinv)
        sp = _sc_edge_s(xwm, row, col)
        cp = _sc_edge_c(xwcs, row, col)
        X = _post(X, sp, cp, xwcs, v, dinv, lam)
    return _dec(X, W_dec.T, b_dec.reshape(1, NCLASS))


# speedup vs baseline: 13.1343x; 1.7473x over previous
import functools

import jax
import jax.numpy as jnp
from jax import lax
from jax.experimental import pallas as pl
from jax.experimental.pallas import tpu as pltpu
from jax.experimental.pallas import tpu_sc as plsc

N = 10000
E = 320000
H = 128
NCLASS = 40

_NTILE = 16
_NW = 32
_NP = 10240
_EP = _NW * _NP
_EPW = _EP // _NW
_RPT = _NP // _NTILE

_KS = 80
_NCHS = _EPW // _KS
_KC = 160
_NCHC = _EPW // _KC
_KD = 640
_NCHD = _EPW // _KD

_mesh = plsc.VectorSubcoreMesh(core_axis_name="c", subcore_axis_name="s")


@functools.partial(
    pl.kernel,
    mesh=_mesh,
    out_type=jax.ShapeDtypeStruct((2, _NP), jnp.float32),
    scratch_types=[
        pltpu.VMEM((_KD,), jnp.int32),
        pltpu.VMEM((_KD,), jnp.float32),
        pltpu.VMEM((_RPT,), jnp.float32),
        pltpu.VMEM_SHARED((_NP,), jnp.float32),
    ],
)
def _sc_degree(col_hbm, out_hbm, idxc, ones, zbuf, acc):
    c = lax.axis_index("c")
    s = lax.axis_index("s")
    wid = c * _NTILE + s

    def fill(i, _):
        ones[pl.ds(i * 16, 16)] = jnp.full((16,), 1.0, jnp.float32)
        zbuf[pl.ds(i * 16, 16)] = jnp.zeros((16,), jnp.float32)
        return 0
    lax.fori_loop(0, _RPT // 16, fill, 0)

    pltpu.sync_copy(zbuf, acc.at[pl.ds(s * _RPT, _RPT)])
    plsc.subcore_barrier()

    base = wid * _EPW

    def chunk(k, _):
        pltpu.sync_copy(col_hbm.at[pl.ds(base + k * _KD, _KD)], idxc)
        pltpu.sync_copy(ones, acc.at[idxc], add=True)
        return 0
    lax.fori_loop(0, _NCHD, chunk, 0)

    plsc.subcore_barrier()
    pltpu.sync_copy(acc.at[pl.ds(s * _RPT, _RPT)],
                    out_hbm.at[c, pl.ds(s * _RPT, _RPT)])


@functools.partial(
    pl.kernel,
    mesh=_mesh,
    out_type=jax.ShapeDtypeStruct((2, _NP, H), jnp.float32),
    scratch_types=[
        pltpu.VMEM((_KS,), jnp.int32),
        pltpu.VMEM((_KS,), jnp.int32),
        pltpu.VMEM((_KS,), jnp.int32),
        pltpu.VMEM((_KS,), jnp.int32),
        pltpu.VMEM((_KS, H), jnp.float32),
        pltpu.VMEM((_KS, H), jnp.float32),
        pltpu.VMEM((_KS, H), jnp.float32),
        pltpu.VMEM((_KS, H), jnp.float32),
        pltpu.VMEM_SHARED((_NP, H), jnp.float32),
        pltpu.SemaphoreType.DMA,
        pltpu.SemaphoreType.DMA,
        pltpu.SemaphoreType.DMA,
        pltpu.SemaphoreType.DMA,
    ],
)
def _sc_edge_s(t_hbm, row_hbm, col_hbm, out_hbm,
               idxr0, idxr1, idxc0, idxc1,
               a0, a1, b0, b1, acc, sa0, sa1, sb0, sb1):
    c = lax.axis_index("c")
    s = lax.axis_index("s")
    wid = c * _NTILE + s
    idxr = (idxr0, idxr1)
    idxc = (idxc0, idxc1)
    a = (a0, a1)
    b = (b0, b1)
    sa = (sa0, sa1)
    sb = (sb0, sb1)

    def zrow(r, _):
        for j in range(H // 16):
            b0[r, pl.ds(j * 16, 16)] = jnp.zeros((16,), jnp.float32)
        return 0
    lax.fori_loop(0, _KS, zrow, 0)
    for t in range(_RPT // _KS):
        pltpu.sync_copy(b0, acc.at[pl.ds(s * _RPT + t * _KS, _KS)])
    plsc.subcore_barrier()

    base = wid * _EPW

    def load_and_fire(k, sl):
        off = base + k * _KS
        pltpu.sync_copy(row_hbm.at[pl.ds(off, _KS)], idxr[sl])
        pltpu.sync_copy(col_hbm.at[pl.ds(off, _KS)], idxc[sl])
        pltpu.async_copy(t_hbm.at[idxr[sl]], a[sl], sa[sl])
        pltpu.async_copy(t_hbm.at[idxc[sl]], b[sl], sb[sl])

    for sl in (0, 1):
        load_and_fire(sl, sl)

    def group(g, _):
        for sl in (0, 1):
            k = 2 * g + sl
            pltpu.make_async_copy(t_hbm.at[idxr[sl]], a[sl], sa[sl]).wait()
            pltpu.make_async_copy(t_hbm.at[idxc[sl]], b[sl], sb[sl]).wait()

            def relu_row(e, _):
                for j in range(H // 16):
                    slc = pl.ds(j * 16, 16)
                    a[sl][e, slc] = jnp.maximum(a[sl][e, slc] - b[sl][e, slc], 0.0)
                return 0
            lax.fori_loop(0, _KS, relu_row, 0)
            pltpu.sync_copy(a[sl], acc.at[idxc[sl]], add=True)

            @pl.when(k + 2 < _NCHS)
            def _():
                load_and_fire(k + 2, sl)
        return 0
    lax.fori_loop(0, _NCHS // 2, group, 0)

    plsc.subcore_barrier()
    pltpu.sync_copy(acc.at[pl.ds(s * _RPT, _RPT)],
                    out_hbm.at[c, pl.ds(s * _RPT, _RPT)])


@functools.partial(
    pl.kernel,
    mesh=_mesh,
    out_type=jax.ShapeDtypeStruct((2, _NP, H), jnp.float32),
    scratch_types=[
        pltpu.VMEM((_KC,), jnp.int32),
        pltpu.VMEM((_KC,), jnp.int32),
        pltpu.VMEM((_KC,), jnp.int32),
        pltpu.VMEM((_KC,), jnp.int32),
        pltpu.VMEM((_KC, H), jnp.float32),
        pltpu.VMEM((_KC, H), jnp.float32),
        pltpu.VMEM_SHARED((_NP, H), jnp.float32),
        pltpu.SemaphoreType.DMA,
        pltpu.SemaphoreType.DMA,
    ],
)
def _sc_edge_c(t_hbm, row_hbm, col_hbm, out_hbm,
               idxr0, idxr1, idxc0, idxc1, a0, a1, acc, sa0, sa1):
    c = lax.axis_index("c")
    s = lax.axis_index("s")
    wid = c * _NTILE + s
    idxr = (idxr0, idxr1)
    idxc = (idxc0, idxc1)
    a = (a0, a1)
    sa = (sa0, sa1)

    def zrow(r, _):
        for j in range(H // 16):
            a0[r, pl.ds(j * 16, 16)] = jnp.zeros((16,), jnp.float32)
        return 0
    lax.fori_loop(0, _KC, zrow, 0)
    for t in range(_RPT // _KC):
        pltpu.sync_copy(a0, acc.at[pl.ds(s * _RPT + t * _KC, _KC)])
    plsc.subcore_barrier()

    base = wid * _EPW

    def load_and_fire(k, sl):
        off = base + k * _KC
        pltpu.sync_copy(row_hbm.at[pl.ds(off, _KC)], idxr[sl])
        pltpu.sync_copy(col_hbm.at[pl.ds(off, _KC)], idxc[sl])
        pltpu.async_copy(t_hbm.at[idxr[sl]], a[sl], sa[sl])

    for sl in (0, 1):
        load_and_fire(sl, sl)

    def group(g, _):
        for sl in (0, 1):
            k = 2 * g + sl
            pltpu.make_async_copy(t_hbm.at[idxr[sl]], a[sl], sa[sl]).wait()
            pltpu.sync_copy(a[sl], acc.at[idxc[sl]], add=True)

            @pl.when(k + 2 < _NCHC)
            def _():
                load_and_fire(k + 2, sl)
        return 0
    lax.fori_loop(0, _NCHC // 2, group, 0)

    plsc.subcore_barrier()
    pltpu.sync_copy(acc.at[pl.ds(s * _RPT, _RPT)],
                    out_hbm.at[c, pl.ds(s * _RPT, _RPT)])


_BR = 1000
_GRID = N // _BR


def _full(shape):
    return pl.BlockSpec(shape, lambda i: tuple(0 for _ in shape))


def _enc_body(x_ref, w_ref, b_ref, o_ref):
    o_ref[...] = jnp.maximum(
        jnp.dot(x_ref[...], w_ref[...], preferred_element_type=jnp.float32)
        + b_ref[...], 0.0)


def _enc(x, wt, b):
    return pl.pallas_call(
        _enc_body,
        grid=(_GRID,),
        in_specs=[pl.BlockSpec((_BR, H), lambda i: (i, 0)),
                  _full((H, H)), _full((1, H))],
        out_specs=pl.BlockSpec((_BR, H), lambda i: (i, 0)),
        out_shape=jax.ShapeDtypeStruct((N, H), jnp.float32),
    )(x, wt, b)


def _pre_body(x_ref, wmc_ref, wrt_ref, bv_ref, dinv_ref,
              xwm_ref, xwcs_ref, v_ref):
    z = jnp.dot(x_ref[...], wmc_ref[...], preferred_element_type=jnp.float32)
    xwm = z[:, :H]
    xwc = z[:, H:]
    xwm_ref[...] = xwm
    xwcs_ref[...] = xwc * dinv_ref[...]
    v_ref[...] = bv_ref[...] - jnp.dot(
        xwc, wrt_ref[...], preferred_element_type=jnp.float32)


def _pre(x, wmc, wrt, bv, dinv):
    return pl.pallas_call(
        _pre_body,
        grid=(_GRID,),
        in_specs=[pl.BlockSpec((_BR, H), lambda i: (i, 0)),
                  _full((H, 2 * H)), _full((H, H)), _full((1, H)),
                  pl.BlockSpec((_BR, 1), lambda i: (i, 0))],
        out_specs=[pl.BlockSpec((_BR, H), lambda i: (i, 0))] * 3,
        out_shape=[jax.ShapeDtypeStruct((N, H), jnp.float32)] * 3,
    )(x, wmc, wrt, bv, dinv)


def _post_body(x_ref, sp_ref, cp_ref, xwcs_ref, v_ref, dinv_ref, lam_ref, o_ref):
    ssum = sp_ref[0] + sp_ref[1]
    csum = cp_ref[0] + cp_ref[1]
    t = jnp.maximum(dinv_ref[...] * (csum + xwcs_ref[...]) + v_ref[...], 0.0)
    o_ref[...] = t + lam_ref[0, 0] * x_ref[...] * ssum


def _post(x, sp, cp, xwcs, v, dinv, lam):
    return pl.pallas_call(
        _post_body,
        grid=(_GRID,),
        in_specs=[pl.BlockSpec((_BR, H), lambda i: (i, 0)),
                  pl.BlockSpec((2, _BR, H), lambda i: (0, i, 0)),
                  pl.BlockSpec((2, _BR, H), lambda i: (0, i, 0)),
                  pl.BlockSpec((_BR, H), lambda i: (i, 0)),
                  pl.BlockSpec((_BR, H), lambda i: (i, 0)),
                  pl.BlockSpec((_BR, 1), lambda i: (i, 0)),
                  _full((1, 1))],
        out_specs=pl.BlockSpec((_BR, H), lambda i: (i, 0)),
        out_shape=jax.ShapeDtypeStruct((N, H), jnp.float32),
    )(x, sp, cp, xwcs, v, dinv, lam)


def _dec_body(x_ref, w_ref, b_ref, o_ref):
    o_ref[...] = jnp.dot(
        x_ref[...], w_ref[...], preferred_element_type=jnp.float32) + b_ref[...]


def _dec(x, wt, b):
    return pl.pallas_call(
        _dec_body,
        grid=(_GRID,),
        in_specs=[pl.BlockSpec((_BR, H), lambda i: (i, 0)),
                  _full((H, NCLASS)), _full((1, NCLASS))],
        out_specs=pl.BlockSpec((_BR, NCLASS), lambda i: (i, 0)),
        out_shape=jax.ShapeDtypeStruct((N, NCLASS), jnp.float32),
    )(x, wt, b)


def kernel(x, edge_index, W_enc, b_enc, W_conv, b_conv, W_res, b_res,
           W_dec, b_dec, weight_mlp, lamda1):
    row = edge_index[0]
    col = edge_index[1]
    npad = _EP - E
    pad_r = jnp.arange(npad, dtype=jnp.int32) % N
    pad_c = (jnp.arange(npad, dtype=jnp.int32) % (_NP - N)) + N
    rowp = jnp.concatenate([row, pad_r])
    colp = jnp.concatenate([col, pad_c])

    degp = _sc_degree(colp)
    deg = degp[0, :N] + degp[1, :N] + 1.0
    dinv = jax.lax.rsqrt(deg).reshape(N, 1)

    X = _enc(x, W_enc.T, b_enc.reshape(1, H))
    wmc = jnp.concatenate([weight_mlp, W_conv.T], axis=1)
    bv = (b_conv - b_res).reshape(1, H)
    lam = lamda1.reshape(1, 1)
    for _ in range(2):
        xwm, xwcs, v = _pre(X, wmc, W_res.T, bv, dinv)
        sp = _sc_edge_s(xwm, rowp, colp)
        cp = _sc_edge_c(xwcs, rowp, colp)
        X = _post(X, sp, cp, xwcs, v, dinv, lam)
    return _dec(X, W_dec.T, b_dec.reshape(1, NCLASS))


# speedup vs baseline: 17.2331x; 1.3121x over previous
import functools

import jax
import jax.numpy as jnp
from jax import lax
from jax.experimental import pallas as pl
from jax.experimental.pallas import tpu as pltpu
from jax.experimental.pallas import tpu_sc as plsc

N = 10000
E = 320000
H = 128
NCLASS = 40

_NTILE = 16
_NW = 32
_NP = 10240
_EP = _NW * _NP
_EPW = _EP // _NW
_RPT = _NP // _NTILE

_KS = 80
_NCHS = _EPW // _KS
_KC = 160
_NCHC = _EPW // _KC
_KD = 640
_NCHD = _EPW // _KD

_mesh = plsc.VectorSubcoreMesh(core_axis_name="c", subcore_axis_name="s")


@functools.partial(
    pl.kernel,
    mesh=_mesh,
    out_type=jax.ShapeDtypeStruct((2, _NP), jnp.float32),
    scratch_types=[
        pltpu.VMEM((_KD,), jnp.int32),
        pltpu.VMEM((_KD,), jnp.float32),
        pltpu.VMEM((_RPT,), jnp.float32),
        pltpu.VMEM_SHARED((_NP,), jnp.float32),
    ],
)
def _sc_degree(col_hbm, out_hbm, idxc, ones, zbuf, acc):
    c = lax.axis_index("c")
    s = lax.axis_index("s")
    wid = c * _NTILE + s

    def fill(i, _):
        ones[pl.ds(i * 16, 16)] = jnp.full((16,), 1.0, jnp.float32)
        zbuf[pl.ds(i * 16, 16)] = jnp.zeros((16,), jnp.float32)
        return 0
    lax.fori_loop(0, _RPT // 16, fill, 0)

    pltpu.sync_copy(zbuf, acc.at[pl.ds(s * _RPT, _RPT)])
    plsc.subcore_barrier()

    base = wid * _EPW

    def chunk(k, _):
        pltpu.sync_copy(col_hbm.at[pl.ds(base + k * _KD, _KD)], idxc)
        pltpu.sync_copy(ones, acc.at[idxc], add=True)
        return 0
    lax.fori_loop(0, _NCHD, chunk, 0)

    plsc.subcore_barrier()
    pltpu.sync_copy(acc.at[pl.ds(s * _RPT, _RPT)],
                    out_hbm.at[c, pl.ds(s * _RPT, _RPT)])


@functools.partial(
    pl.kernel,
    mesh=_mesh,
    out_type=jax.ShapeDtypeStruct((2, _NP, H), jnp.float32),
    scratch_types=[
        pltpu.VMEM((_KS,), jnp.int32),
        pltpu.VMEM((_KS,), jnp.int32),
        pltpu.VMEM((_KS,), jnp.int32),
        pltpu.VMEM((_KS,), jnp.int32),
        pltpu.VMEM((_KS,), jnp.int32),
        pltpu.VMEM((_KS,), jnp.int32),
        pltpu.VMEM((_KS,), jnp.int32),
        pltpu.VMEM((_KS,), jnp.int32),
        pltpu.VMEM((_KS, H), jnp.float32),
        pltpu.VMEM((_KS, H), jnp.float32),
        pltpu.VMEM((_KS, H), jnp.float32),
        pltpu.VMEM((_KS, H), jnp.float32),
        pltpu.VMEM_SHARED((_NP, H), jnp.float32),
        pltpu.SemaphoreType.DMA,
        pltpu.SemaphoreType.DMA,
        pltpu.SemaphoreType.DMA,
        pltpu.SemaphoreType.DMA,
        pltpu.SemaphoreType.DMA,
        pltpu.SemaphoreType.DMA,
        pltpu.SemaphoreType.DMA,
        pltpu.SemaphoreType.DMA,
    ],
)
def _sc_edge_s(t_hbm, row_hbm, col_hbm, out_hbm,
               idxr0, idxr1, idxr2, idxr3, idxc0, idxc1, idxc2, idxc3,
               a0, a1, b0, b1, acc,
               sa0, sa1, sb0, sb1, si0, si1, si2, si3):
    c = lax.axis_index("c")
    s = lax.axis_index("s")
    wid = c * _NTILE + s
    idxr = (idxr0, idxr1, idxr2, idxr3)
    idxc = (idxc0, idxc1, idxc2, idxc3)
    a = (a0, a1)
    b = (b0, b1)
    sa = (sa0, sa1)
    sb = (sb0, sb1)
    si = (si0, si1, si2, si3)

    def zrow(r, _):
        for j in range(H // 16):
            b0[r, pl.ds(j * 16, 16)] = jnp.zeros((16,), jnp.float32)
        return 0
    lax.fori_loop(0, _KS, zrow, 0)
    for t in range(_RPT // _KS):
        pltpu.sync_copy(b0, acc.at[pl.ds(s * _RPT + t * _KS, _KS)])
    plsc.subcore_barrier()

    base = wid * _EPW

    def load_idx(k, sl):
        off = base + k * _KS
        pltpu.async_copy(row_hbm.at[pl.ds(off, _KS)], idxr[sl], si[sl])
        pltpu.async_copy(col_hbm.at[pl.ds(off, _KS)], idxc[sl], si[sl])

    def wait_idx(sl):
        pltpu.make_async_copy(row_hbm.at[pl.ds(0, _KS)], idxr[sl], si[sl]).wait()
        pltpu.make_async_copy(col_hbm.at[pl.ds(0, _KS)], idxc[sl], si[sl]).wait()

    def fire(sl, bs):
        pltpu.async_copy(t_hbm.at[idxr[sl]], a[bs], sa[bs])
        pltpu.async_copy(t_hbm.at[idxc[sl]], b[bs], sb[bs])

    for sl in (0, 1, 2, 3):
        load_idx(sl, sl)
    for sl in (0, 1):
        wait_idx(sl)
        fire(sl, sl)

    def group(g, _):
        for sl in (0, 1, 2, 3):
            k = 4 * g + sl
            bs = sl % 2
            pltpu.make_async_copy(t_hbm.at[idxr[sl]], a[bs], sa[bs]).wait()
            pltpu.make_async_copy(t_hbm.at[idxc[sl]], b[bs], sb[bs]).wait()

            def relu_row(e, _):
                for j in range(H // 16):
                    slc = pl.ds(j * 16, 16)
                    a[bs][e, slc] = jnp.maximum(a[bs][e, slc] - b[bs][e, slc], 0.0)
                return 0
            lax.fori_loop(0, _KS, relu_row, 0)
            pltpu.sync_copy(a[bs], acc.at[idxc[sl]], add=True)

            @pl.when(k + 4 < _NCHS)
            def _():
                load_idx(k + 4, sl)

            @pl.when(k + 2 < _NCHS)
            def _():
                wait_idx((sl + 2) % 4)
                fire((sl + 2) % 4, bs)
        return 0
    lax.fori_loop(0, _NCHS // 4, group, 0)

    plsc.subcore_barrier()
    pltpu.sync_copy(acc.at[pl.ds(s * _RPT, _RPT)],
                    out_hbm.at[c, pl.ds(s * _RPT, _RPT)])


@functools.partial(
    pl.kernel,
    mesh=_mesh,
    out_type=jax.ShapeDtypeStruct((2, _NP, H), jnp.float32),
    scratch_types=[
        pltpu.VMEM((_KC,), jnp.int32),
        pltpu.VMEM((_KC,), jnp.int32),
        pltpu.VMEM((_KC,), jnp.int32),
        pltpu.VMEM((_KC,), jnp.int32),
        pltpu.VMEM((_KC,), jnp.int32),
        pltpu.VMEM((_KC,), jnp.int32),
        pltpu.VMEM((_KC,), jnp.int32),
        pltpu.VMEM((_KC,), jnp.int32),
        pltpu.VMEM((_KC, H), jnp.float32),
        pltpu.VMEM((_KC, H), jnp.float32),
        pltpu.VMEM_SHARED((_NP, H), jnp.float32),
        pltpu.SemaphoreType.DMA,
        pltpu.SemaphoreType.DMA,
        pltpu.SemaphoreType.DMA,
        pltpu.SemaphoreType.DMA,
        pltpu.SemaphoreType.DMA,
        pltpu.SemaphoreType.DMA,
    ],
)
def _sc_edge_c(t_hbm, row_hbm, col_hbm, out_hbm,
               idxr0, idxr1, idxr2, idxr3, idxc0, idxc1, idxc2, idxc3,
               a0, a1, acc, sa0, sa1, si0, si1, si2, si3):
    c = lax.axis_index("c")
    s = lax.axis_index("s")
    wid = c * _NTILE + s
    idxr = (idxr0, idxr1, idxr2, idxr3)
    idxc = (idxc0, idxc1, idxc2, idxc3)
    a = (a0, a1)
    sa = (sa0, sa1)
    si = (si0, si1, si2, si3)

    def zrow(r, _):
        for j in range(H // 16):
            a0[r, pl.ds(j * 16, 16)] = jnp.zeros((16,), jnp.float32)
        return 0
    lax.fori_loop(0, _KC, zrow, 0)
    for t in range(_RPT // _KC):
        pltpu.sync_copy(a0, acc.at[pl.ds(s * _RPT + t * _KC, _KC)])
    plsc.subcore_barrier()

    base = wid * _EPW

    def load_idx(k, sl):
        off = base + k * _KC
        pltpu.async_copy(row_hbm.at[pl.ds(off, _KC)], idxr[sl], si[sl])
        pltpu.async_copy(col_hbm.at[pl.ds(off, _KC)], idxc[sl], si[sl])

    def wait_idx(sl):
        pltpu.make_async_copy(row_hbm.at[pl.ds(0, _KC)], idxr[sl], si[sl]).wait()
        pltpu.make_async_copy(col_hbm.at[pl.ds(0, _KC)], idxc[sl], si[sl]).wait()

    def fire(sl, bs):
        pltpu.async_copy(t_hbm.at[idxr[sl]], a[bs], sa[bs])

    for sl in (0, 1, 2, 3):
        load_idx(sl, sl)
    for sl in (0, 1):
        wait_idx(sl)
        fire(sl, sl)

    def group(g, _):
        for sl in (0, 1, 2, 3):
            k = 4 * g + sl
            bs = sl % 2
            pltpu.make_async_copy(t_hbm.at[idxr[sl]], a[bs], sa[bs]).wait()
            pltpu.sync_copy(a[bs], acc.at[idxc[sl]], add=True)

            @pl.when(k + 4 < _NCHC)
            def _():
                load_idx(k + 4, sl)

            @pl.when(k + 2 < _NCHC)
            def _():
                wait_idx((sl + 2) % 4)
                fire((sl + 2) % 4, bs)
        return 0
    lax.fori_loop(0, _NCHC // 4, group, 0)

    plsc.subcore_barrier()
    pltpu.sync_copy(acc.at[pl.ds(s * _RPT, _RPT)],
                    out_hbm.at[c, pl.ds(s * _RPT, _RPT)])


_BR = 1000
_GRID = N // _BR


def _full(shape):
    return pl.BlockSpec(shape, lambda i: tuple(0 for _ in shape))


def _enc_body(x_ref, w_ref, b_ref, o_ref):
    o_ref[...] = jnp.maximum(
        jnp.dot(x_ref[...], w_ref[...], preferred_element_type=jnp.float32)
        + b_ref[...], 0.0)


def _enc(x, wt, b):
    return pl.pallas_call(
        _enc_body,
        grid=(_GRID,),
        in_specs=[pl.BlockSpec((_BR, H), lambda i: (i, 0)),
                  _full((H, H)), _full((1, H))],
        out_specs=pl.BlockSpec((_BR, H), lambda i: (i, 0)),
        out_shape=jax.ShapeDtypeStruct((N, H), jnp.float32),
    )(x, wt, b)


def _pre_body(x_ref, wmc_ref, wrt_ref, bv_ref, dinv_ref,
              xwm_ref, xwcs_ref, v_ref):
    z = jnp.dot(x_ref[...], wmc_ref[...], preferred_element_type=jnp.float32)
    xwm = z[:, :H]
    xwc = z[:, H:]
    xwm_ref[...] = xwm
    xwcs_ref[...] = xwc * dinv_ref[...]
    v_ref[...] = bv_ref[...] - jnp.dot(
        xwc, wrt_ref[...], preferred_element_type=jnp.float32)


def _pre(x, wmc, wrt, bv, dinv):
    return pl.pallas_call(
        _pre_body,
        grid=(_GRID,),
        in_specs=[pl.BlockSpec((_BR, H), lambda i: (i, 0)),
                  _full((H, 2 * H)), _full((H, H)), _full((1, H)),
                  pl.BlockSpec((_BR, 1), lambda i: (i, 0))],
        out_specs=[pl.BlockSpec((_BR, H), lambda i: (i, 0))] * 3,
        out_shape=[jax.ShapeDtypeStruct((N, H), jnp.float32)] * 3,
    )(x, wmc, wrt, bv, dinv)


def _post_body(x_ref, sp_ref, cp_ref, xwcs_ref, v_ref, dinv_ref, lam_ref, o_ref):
    ssum = sp_ref[0] + sp_ref[1]
    csum = cp_ref[0] + cp_ref[1]
    t = jnp.maximum(dinv_ref[...] * (csum + xwcs_ref[...]) + v_ref[...], 0.0)
    o_ref[...] = t + lam_ref[0, 0] * x_ref[...] * ssum


def _post(x, sp, cp, xwcs, v, dinv, lam):
    return pl.pallas_call(
        _post_body,
        grid=(_GRID,),
        in_specs=[pl.BlockSpec((_BR, H), lambda i: (i, 0)),
                  pl.BlockSpec((2, _BR, H), lambda i: (0, i, 0)),
                  pl.BlockSpec((2, _BR, H), lambda i: (0, i, 0)),
                  pl.BlockSpec((_BR, H), lambda i: (i, 0)),
                  pl.BlockSpec((_BR, H), lambda i: (i, 0)),
                  pl.BlockSpec((_BR, 1), lambda i: (i, 0)),
                  _full((1, 1))],
        out_specs=pl.BlockSpec((_BR, H), lambda i: (i, 0)),
        out_shape=jax.ShapeDtypeStruct((N, H), jnp.float32),
    )(x, sp, cp, xwcs, v, dinv, lam)


def _dec_body(x_ref, w_ref, b_ref, o_ref):
    o_ref[...] = jnp.dot(
        x_ref[...], w_ref[...], preferred_element_type=jnp.float32) + b_ref[...]


def _dec(x, wt, b):
    return pl.pallas_call(
        _dec_body,
        grid=(_GRID,),
        in_specs=[pl.BlockSpec((_BR, H), lambda i: (i, 0)),
                  _full((H, NCLASS)), _full((1, NCLASS))],
        out_specs=pl.BlockSpec((_BR, NCLASS), lambda i: (i, 0)),
        out_shape=jax.ShapeDtypeStruct((N, NCLASS), jnp.float32),
    )(x, wt, b)


def kernel(x, edge_index, W_enc, b_enc, W_conv, b_conv, W_res, b_res,
           W_dec, b_dec, weight_mlp, lamda1):
    row = edge_index[0]
    col = edge_index[1]
    npad = _EP - E
    pad_r = jnp.arange(npad, dtype=jnp.int32) % N
    pad_c = (jnp.arange(npad, dtype=jnp.int32) % (_NP - N)) + N
    rowp = jnp.concatenate([row, pad_r])
    colp = jnp.concatenate([col, pad_c])

    degp = _sc_degree(colp)
    deg = degp[0, :N] + degp[1, :N] + 1.0
    dinv = jax.lax.rsqrt(deg).reshape(N, 1)

    X = _enc(x, W_enc.T, b_enc.reshape(1, H))
    wmc = jnp.concatenate([weight_mlp, W_conv.T], axis=1)
    bv = (b_conv - b_res).reshape(1, H)
    lam = lamda1.reshape(1, 1)
    for _ in range(2):
        xwm, xwcs, v = _pre(X, wmc, W_res.T, bv, dinv)
        sp = _sc_edge_s(xwm, rowp, colp)
        cp = _sc_edge_c(xwcs, rowp, colp)
        X = _post(X, sp, cp, xwcs, v, dinv, lam)
    return _dec(X, W_dec.T, b_dec.reshape(1, NCLASS))


# speedup vs baseline: 17.6423x; 1.0237x over previous
import functools

import jax
import jax.numpy as jnp
from jax import lax
from jax.experimental import pallas as pl
from jax.experimental.pallas import tpu as pltpu
from jax.experimental.pallas import tpu_sc as plsc

N = 10000
E = 320000
H = 128
NCLASS = 40

_NTILE = 16
_NW = 32
_NP = 10240
_EP = _NW * _NP
_EPW = _EP // _NW
_RPT = _NP // _NTILE

_KS = 80
_NCHS = _EPW // _KS
_KC = 160
_NCHC = _EPW // _KC
_KD = 640
_NCHD = _EPW // _KD

_mesh = plsc.VectorSubcoreMesh(core_axis_name="c", subcore_axis_name="s")


@functools.partial(
    pl.kernel,
    mesh=_mesh,
    out_type=jax.ShapeDtypeStruct((2, _NP), jnp.float32),
    scratch_types=[
        pltpu.VMEM((_KD,), jnp.int32),
        pltpu.VMEM((_KD,), jnp.float32),
        pltpu.VMEM((_RPT,), jnp.float32),
        pltpu.VMEM_SHARED((_NP,), jnp.float32),
    ],
)
def _sc_degree(col_hbm, out_hbm, idxc, ones, zbuf, acc):
    c = lax.axis_index("c")
    s = lax.axis_index("s")
    wid = c * _NTILE + s

    def fill(i, _):
        ones[pl.ds(i * 16, 16)] = jnp.full((16,), 1.0, jnp.float32)
        zbuf[pl.ds(i * 16, 16)] = jnp.zeros((16,), jnp.float32)
        return 0
    lax.fori_loop(0, _RPT // 16, fill, 0)

    pltpu.sync_copy(zbuf, acc.at[pl.ds(s * _RPT, _RPT)])
    plsc.subcore_barrier()

    base = wid * _EPW

    def chunk(k, _):
        pltpu.sync_copy(col_hbm.at[pl.ds(base + k * _KD, _KD)], idxc)
        pltpu.sync_copy(ones, acc.at[idxc], add=True)
        return 0
    lax.fori_loop(0, _NCHD, chunk, 0)

    plsc.subcore_barrier()
    pltpu.sync_copy(acc.at[pl.ds(s * _RPT, _RPT)],
                    out_hbm.at[c, pl.ds(s * _RPT, _RPT)])


@functools.partial(
    pl.kernel,
    mesh=_mesh,
    out_type=jax.ShapeDtypeStruct((2, _NP, H), jnp.float32),
    scratch_types=[
        pltpu.VMEM((_KS,), jnp.int32),
        pltpu.VMEM((_KS,), jnp.int32),
        pltpu.VMEM((_KS,), jnp.int32),
        pltpu.VMEM((_KS,), jnp.int32),
        pltpu.VMEM((_KS,), jnp.int32),
        pltpu.VMEM((_KS,), jnp.int32),
        pltpu.VMEM((_KS,), jnp.int32),
        pltpu.VMEM((_KS,), jnp.int32),
        pltpu.VMEM((_KS, H), jnp.float32),
        pltpu.VMEM((_KS, H), jnp.float32),
        pltpu.VMEM((_KS, H), jnp.float32),
        pltpu.VMEM((_KS, H), jnp.float32),
        pltpu.VMEM_SHARED((_NP, H), jnp.float32),
        pltpu.SemaphoreType.DMA,
        pltpu.SemaphoreType.DMA,
        pltpu.SemaphoreType.DMA,
        pltpu.SemaphoreType.DMA,
        pltpu.SemaphoreType.DMA,
        pltpu.SemaphoreType.DMA,
        pltpu.SemaphoreType.DMA,
        pltpu.SemaphoreType.DMA,
    ],
)
def _sc_edge_s(t_hbm, row_hbm, col_hbm, out_hbm,
               idxr0, idxr1, idxr2, idxr3, idxc0, idxc1, idxc2, idxc3,
               a0, a1, b0, b1, acc,
               sa0, sa1, sb0, sb1, si0, si1, si2, si3):
    c = lax.axis_index("c")
    s = lax.axis_index("s")
    wid = c * _NTILE + s
    idxr = (idxr0, idxr1, idxr2, idxr3)
    idxc = (idxc0, idxc1, idxc2, idxc3)
    a = (a0, a1)
    b = (b0, b1)
    sa = (sa0, sa1)
    sb = (sb0, sb1)
    si = (si0, si1, si2, si3)

    def zrow(r, _):
        for j in range(H // 16):
            b0[r, pl.ds(j * 16, 16)] = jnp.zeros((16,), jnp.float32)
        return 0
    lax.fori_loop(0, _KS, zrow, 0)
    for t in range(_RPT // _KS):
        pltpu.sync_copy(b0, acc.at[pl.ds(s * _RPT + t * _KS, _KS)])
    plsc.subcore_barrier()

    base = wid * _EPW

    def load_idx(k, sl):
        off = base + k * _KS
        pltpu.async_copy(row_hbm.at[pl.ds(off, _KS)], idxr[sl], si[sl])
        pltpu.async_copy(col_hbm.at[pl.ds(off, _KS)], idxc[sl], si[sl])

    def wait_idx(sl):
        pltpu.make_async_copy(row_hbm.at[pl.ds(0, _KS)], idxr[sl], si[sl]).wait()
        pltpu.make_async_copy(col_hbm.at[pl.ds(0, _KS)], idxc[sl], si[sl]).wait()

    def fire(sl, bs):
        pltpu.async_copy(t_hbm.at[idxr[sl]], a[bs], sa[bs])
        pltpu.async_copy(t_hbm.at[idxc[sl]], b[bs], sb[bs])

    for sl in (0, 1, 2, 3):
        load_idx(sl, sl)
    for sl in (0, 1):
        wait_idx(sl)
        fire(sl, sl)

    def group(g, _):
        for sl in (0, 1, 2, 3):
            k = 4 * g + sl
            bs = sl % 2
            pltpu.make_async_copy(t_hbm.at[idxr[sl]], a[bs], sa[bs]).wait()
            pltpu.make_async_copy(t_hbm.at[idxc[sl]], b[bs], sb[bs]).wait()

            def relu_row(e, _):
                for j in range(H // 16):
                    slc = pl.ds(j * 16, 16)
                    a[bs][e, slc] = jnp.maximum(a[bs][e, slc] - b[bs][e, slc], 0.0)
                return 0
            lax.fori_loop(0, _KS, relu_row, 0)
            pltpu.sync_copy(a[bs], acc.at[idxc[sl]], add=True)

            @pl.when(k + 4 < _NCHS)
            def _():
                load_idx(k + 4, sl)

            @pl.when(k + 2 < _NCHS)
            def _():
                wait_idx((sl + 2) % 4)
                fire((sl + 2) % 4, bs)
        return 0
    lax.fori_loop(0, _NCHS // 4, group, 0)

    plsc.subcore_barrier()
    pltpu.sync_copy(acc.at[pl.ds(s * _RPT, _RPT)],
                    out_hbm.at[c, pl.ds(s * _RPT, _RPT)])


@functools.partial(
    pl.kernel,
    mesh=_mesh,
    out_type=jax.ShapeDtypeStruct((2, _NP, H), jnp.float32),
    scratch_types=[
        pltpu.VMEM((_KC,), jnp.int32),
        pltpu.VMEM((_KC,), jnp.int32),
        pltpu.VMEM((_KC,), jnp.int32),
        pltpu.VMEM((_KC,), jnp.int32),
        pltpu.VMEM((_KC,), jnp.int32),
        pltpu.VMEM((_KC,), jnp.int32),
        pltpu.VMEM((_KC,), jnp.int32),
        pltpu.VMEM((_KC,), jnp.int32),
        pltpu.VMEM((_KC, H), jnp.float32),
        pltpu.VMEM((_KC, H), jnp.float32),
        pltpu.VMEM_SHARED((_NP, H), jnp.float32),
        pltpu.SemaphoreType.DMA,
        pltpu.SemaphoreType.DMA,
        pltpu.SemaphoreType.DMA,
        pltpu.SemaphoreType.DMA,
        pltpu.SemaphoreType.DMA,
        pltpu.SemaphoreType.DMA,
    ],
)
def _sc_edge_c(t_hbm, row_hbm, col_hbm, out_hbm,
               idxr0, idxr1, idxr2, idxr3, idxc0, idxc1, idxc2, idxc3,
               a0, a1, acc, sa0, sa1, si0, si1, si2, si3):
    c = lax.axis_index("c")
    s = lax.axis_index("s")
    wid = c * _NTILE + s
    idxr = (idxr0, idxr1, idxr2, idxr3)
    idxc = (idxc0, idxc1, idxc2, idxc3)
    a = (a0, a1)
    sa = (sa0, sa1)
    si = (si0, si1, si2, si3)

    def zrow(r, _):
        for j in range(H // 16):
            a0[r, pl.ds(j * 16, 16)] = jnp.zeros((16,), jnp.float32)
        return 0
    lax.fori_loop(0, _KC, zrow, 0)
    for t in range(_RPT // _KC):
        pltpu.sync_copy(a0, acc.at[pl.ds(s * _RPT + t * _KC, _KC)])
    plsc.subcore_barrier()

    base = wid * _EPW

    def load_idx(k, sl):
        off = base + k * _KC
        pltpu.async_copy(row_hbm.at[pl.ds(off, _KC)], idxr[sl], si[sl])
        pltpu.async_copy(col_hbm.at[pl.ds(off, _KC)], idxc[sl], si[sl])

    def wait_idx(sl):
        pltpu.make_async_copy(row_hbm.at[pl.ds(0, _KC)], idxr[sl], si[sl]).wait()
        pltpu.make_async_copy(col_hbm.at[pl.ds(0, _KC)], idxc[sl], si[sl]).wait()

    def fire(sl, bs):
        pltpu.async_copy(t_hbm.at[idxr[sl]], a[bs], sa[bs])

    for sl in (0, 1, 2, 3):
        load_idx(sl, sl)
    for sl in (0, 1):
        wait_idx(sl)
        fire(sl, sl)

    def group(g, _):
        for sl in (0, 1, 2, 3):
            k = 4 * g + sl
            bs = sl % 2
            pltpu.make_async_copy(t_hbm.at[idxr[sl]], a[bs], sa[bs]).wait()
            pltpu.sync_copy(a[bs], acc.at[idxc[sl]], add=True)

            @pl.when(k + 4 < _NCHC)
            def _():
                load_idx(k + 4, sl)

            @pl.when(k + 2 < _NCHC)
            def _():
                wait_idx((sl + 2) % 4)
                fire((sl + 2) % 4, bs)
        return 0
    lax.fori_loop(0, _NCHC // 4, group, 0)

    plsc.subcore_barrier()
    pltpu.sync_copy(acc.at[pl.ds(s * _RPT, _RPT)],
                    out_hbm.at[c, pl.ds(s * _RPT, _RPT)])


_BR = 1000
_GRID = N // _BR


def _full(shape):
    return pl.BlockSpec(shape, lambda i: tuple(0 for _ in shape))


def _post_block(x, sp, cp, xwcs, v, dinv, lam):
    ssum = sp[0] + sp[1]
    csum = cp[0] + cp[1]
    t = jnp.maximum(dinv * (csum + xwcs) + v, 0.0)
    return t + lam * x * ssum


def _pre_block(x, wmc, wrt, bv, dinv):
    z = jnp.dot(x, wmc, preferred_element_type=jnp.float32)
    xwc = z[:, H:]
    v = bv - jnp.dot(xwc, wrt, preferred_element_type=jnp.float32)
    return z[:, :H], xwc * dinv, v


def _encpre_body(x_ref, wet_ref, be_ref, wmc_ref, wrt_ref, bv_ref, dinv_ref,
                 x_out, xwm_ref, xwcs_ref, v_ref):
    xx = jnp.maximum(
        jnp.dot(x_ref[...], wet_ref[...], preferred_element_type=jnp.float32)
        + be_ref[...], 0.0)
    x_out[...] = xx
    xwm, xwcs, v = _pre_block(xx, wmc_ref[...], wrt_ref[...], bv_ref[...],
                              dinv_ref[...])
    xwm_ref[...] = xwm
    xwcs_ref[...] = xwcs
    v_ref[...] = v


def _encpre(x, wet, be, wmc, wrt, bv, dinv):
    return pl.pallas_call(
        _encpre_body,
        grid=(_GRID,),
        in_specs=[pl.BlockSpec((_BR, H), lambda i: (i, 0)),
                  _full((H, H)), _full((1, H)),
                  _full((H, 2 * H)), _full((H, H)), _full((1, H)),
                  pl.BlockSpec((_BR, 1), lambda i: (i, 0))],
        out_specs=[pl.BlockSpec((_BR, H), lambda i: (i, 0))] * 4,
        out_shape=[jax.ShapeDtypeStruct((N, H), jnp.float32)] * 4,
    )(x, wet, be, wmc, wrt, bv, dinv)


def _postpre_body(x_ref, sp_ref, cp_ref, xwcs_ref, v_ref, dinv_ref, lam_ref,
                  wmc_ref, wrt_ref, bv_ref,
                  x_out, xwm_ref, xwcs_out, v_out):
    xn = _post_block(x_ref[...], sp_ref, cp_ref, xwcs_ref[...], v_ref[...],
                     dinv_ref[...], lam_ref[0, 0])
    x_out[...] = xn
    xwm, xwcs, v = _pre_block(xn, wmc_ref[...], wrt_ref[...], bv_ref[...],
                              dinv_ref[...])
    xwm_ref[...] = xwm
    xwcs_out[...] = xwcs
    v_out[...] = v


def _postpre(x, sp, cp, xwcs, v, dinv, lam, wmc, wrt, bv):
    return pl.pallas_call(
        _postpre_body,
        grid=(_GRID,),
        in_specs=[pl.BlockSpec((_BR, H), lambda i: (i, 0)),
                  pl.BlockSpec((2, _BR, H), lambda i: (0, i, 0)),
                  pl.BlockSpec((2, _BR, H), lambda i: (0, i, 0)),
                  pl.BlockSpec((_BR, H), lambda i: (i, 0)),
                  pl.BlockSpec((_BR, H), lambda i: (i, 0)),
                  pl.BlockSpec((_BR, 1), lambda i: (i, 0)),
                  _full((1, 1)),
                  _full((H, 2 * H)), _full((H, H)), _full((1, H))],
        out_specs=[pl.BlockSpec((_BR, H), lambda i: (i, 0))] * 4,
        out_shape=[jax.ShapeDtypeStruct((N, H), jnp.float32)] * 4,
    )(x, sp, cp, xwcs, v, dinv, lam, wmc, wrt, bv)


def _postdec_body(x_ref, sp_ref, cp_ref, xwcs_ref, v_ref, dinv_ref, lam_ref,
                  wdt_ref, bd_ref, o_ref):
    xn = _post_block(x_ref[...], sp_ref, cp_ref, xwcs_ref[...], v_ref[...],
                     dinv_ref[...], lam_ref[0, 0])
    o_ref[...] = jnp.dot(
        xn, wdt_ref[...], preferred_element_type=jnp.float32) + bd_ref[...]


def _postdec(x, sp, cp, xwcs, v, dinv, lam, wdt, bd):
    return pl.pallas_call(
        _postdec_body,
        grid=(_GRID,),
        in_specs=[pl.BlockSpec((_BR, H), lambda i: (i, 0)),
                  pl.BlockSpec((2, _BR, H), lambda i: (0, i, 0)),
                  pl.BlockSpec((2, _BR, H), lambda i: (0, i, 0)),
                  pl.BlockSpec((_BR, H), lambda i: (i, 0)),
                  pl.BlockSpec((_BR, H), lambda i: (i, 0)),
                  pl.BlockSpec((_BR, 1), lambda i: (i, 0)),
                  _full((1, 1)),
                  _full((H, NCLASS)), _full((1, NCLASS))],
        out_specs=pl.BlockSpec((_BR, NCLASS), lambda i: (i, 0)),
        out_shape=jax.ShapeDtypeStruct((N, NCLASS), jnp.float32),
    )(x, sp, cp, xwcs, v, dinv, lam, wdt, bd)


def kernel(x, edge_index, W_enc, b_enc, W_conv, b_conv, W_res, b_res,
           W_dec, b_dec, weight_mlp, lamda1):
    row = edge_index[0]
    col = edge_index[1]
    npad = _EP - E
    pad_r = jnp.arange(npad, dtype=jnp.int32) % N
    pad_c = (jnp.arange(npad, dtype=jnp.int32) % (_NP - N)) + N
    rowp = jnp.concatenate([row, pad_r])
    colp = jnp.concatenate([col, pad_c])

    degp = _sc_degree(colp)
    deg = degp[0, :N] + degp[1, :N] + 1.0
    dinv = jax.lax.rsqrt(deg).reshape(N, 1)

    wmc = jnp.concatenate([weight_mlp, W_conv.T], axis=1)
    bv = (b_conv - b_res).reshape(1, H)
    lam = lamda1.reshape(1, 1)

    X, xwm, xwcs, v = _encpre(x, W_enc.T, b_enc.reshape(1, H),
                              wmc, W_res.T, bv, dinv)
    sp = _sc_edge_s(xwm, rowp, colp)
    cp = _sc_edge_c(xwcs, rowp, colp)
    X, xwm, xwcs, v = _postpre(X, sp, cp, xwcs, v, dinv, lam,
                               wmc, W_res.T, bv)
    sp = _sc_edge_s(xwm, rowp, colp)
    cp = _sc_edge_c(xwcs, rowp, colp)
    return _postdec(X, sp, cp, xwcs, v, dinv, lam,
                    W_dec.T, b_dec.reshape(1, NCLASS))


# speedup vs baseline: 17.7691x; 1.0072x over previous
import functools

import jax
import jax.numpy as jnp
from jax import lax
from jax.experimental import pallas as pl
from jax.experimental.pallas import tpu as pltpu
from jax.experimental.pallas import tpu_sc as plsc

N = 10000
E = 320000
H = 128
NCLASS = 40

_NTILE = 16
_NW = 32
_NP = 10240
_EP = _NW * _NP
_EPW = _EP // _NW
_RPT = _NP // _NTILE

_KS = 80
_NCHS = _EPW // _KS
_KC = 160
_NCHC = _EPW // _KC
_KD = 640
_NCHD = _EPW // _KD

_mesh = plsc.VectorSubcoreMesh(core_axis_name="c", subcore_axis_name="s")


@functools.partial(
    pl.kernel,
    mesh=_mesh,
    out_type=jax.ShapeDtypeStruct((2, _NP), jnp.float32),
    scratch_types=[
        pltpu.VMEM((_KD,), jnp.int32),
        pltpu.VMEM((_KD,), jnp.float32),
        pltpu.VMEM((_RPT,), jnp.float32),
        pltpu.VMEM_SHARED((_NP,), jnp.float32),
    ],
)
def _sc_degree(col_hbm, out_hbm, idxc, ones, zbuf, acc):
    c = lax.axis_index("c")
    s = lax.axis_index("s")
    wid = c * _NTILE + s

    def fill(i, _):
        ones[pl.ds(i * 16, 16)] = jnp.full((16,), 1.0, jnp.float32)
        zbuf[pl.ds(i * 16, 16)] = jnp.zeros((16,), jnp.float32)
        return 0
    lax.fori_loop(0, _RPT // 16, fill, 0)

    pltpu.sync_copy(zbuf, acc.at[pl.ds(s * _RPT, _RPT)])
    plsc.subcore_barrier()

    base = wid * _EPW

    def chunk(k, _):
        pltpu.sync_copy(col_hbm.at[pl.ds(base + k * _KD, _KD)], idxc)
        pltpu.sync_copy(ones, acc.at[idxc], add=True)
        return 0
    lax.fori_loop(0, _NCHD, chunk, 0)

    plsc.subcore_barrier()
    pltpu.sync_copy(acc.at[pl.ds(s * _RPT, _RPT)],
                    out_hbm.at[c, pl.ds(s * _RPT, _RPT)])


@functools.partial(
    pl.kernel,
    mesh=_mesh,
    out_type=jax.ShapeDtypeStruct((2, _NP, H), jnp.float32),
    scratch_types=[
        pltpu.VMEM((_KS,), jnp.int32),
        pltpu.VMEM((_KS,), jnp.int32),
        pltpu.VMEM((_KS,), jnp.int32),
        pltpu.VMEM((_KS,), jnp.int32),
        pltpu.VMEM((_KS,), jnp.int32),
        pltpu.VMEM((_KS,), jnp.int32),
        pltpu.VMEM((_KS,), jnp.int32),
        pltpu.VMEM((_KS,), jnp.int32),
        pltpu.VMEM((_KS, H), jnp.float32),
        pltpu.VMEM((_KS, H), jnp.float32),
        pltpu.VMEM((_KS, H), jnp.float32),
        pltpu.VMEM((_KS, H), jnp.float32),
        pltpu.VMEM_SHARED((_NP, H), jnp.float32),
        pltpu.SemaphoreType.DMA,
        pltpu.SemaphoreType.DMA,
        pltpu.SemaphoreType.DMA,
        pltpu.SemaphoreType.DMA,
        pltpu.SemaphoreType.DMA,
        pltpu.SemaphoreType.DMA,
        pltpu.SemaphoreType.DMA,
        pltpu.SemaphoreType.DMA,
    ],
)
def _sc_edge_s(t_hbm, row_hbm, col_hbm, out_hbm,
               idxr0, idxr1, idxr2, idxr3, idxc0, idxc1, idxc2, idxc3,
               a0, a1, b0, b1, acc,
               sa0, sa1, sb0, sb1, si0, si1, si2, si3):
    c = lax.axis_index("c")
    s = lax.axis_index("s")
    wid = c * _NTILE + s
    idxr = (idxr0, idxr1, idxr2, idxr3)
    idxc = (idxc0, idxc1, idxc2, idxc3)
    a = (a0, a1)
    b = (b0, b1)
    sa = (sa0, sa1)
    sb = (sb0, sb1)
    si = (si0, si1, si2, si3)

    def zrow(r, _):
        for j in range(H // 16):
            b0[r, pl.ds(j * 16, 16)] = jnp.zeros((16,), jnp.float32)
        return 0
    lax.fori_loop(0, _KS, zrow, 0)
    for t in range(_RPT // _KS):
        pltpu.sync_copy(b0, acc.at[pl.ds(s * _RPT + t * _KS, _KS)])
    plsc.subcore_barrier()

    base = wid * _EPW

    def load_idx(k, sl):
        off = base + k * _KS
        pltpu.async_copy(row_hbm.at[pl.ds(off, _KS)], idxr[sl], si[sl])
        pltpu.async_copy(col_hbm.at[pl.ds(off, _KS)], idxc[sl], si[sl])

    def wait_idx(sl):
        pltpu.make_async_copy(row_hbm.at[pl.ds(0, _KS)], idxr[sl], si[sl]).wait()
        pltpu.make_async_copy(col_hbm.at[pl.ds(0, _KS)], idxc[sl], si[sl]).wait()

    def fire(sl, bs):
        pltpu.async_copy(t_hbm.at[idxr[sl]], a[bs], sa[bs])
        pltpu.async_copy(t_hbm.at[idxc[sl]], b[bs], sb[bs])

    for sl in (0, 1, 2, 3):
        load_idx(sl, sl)
    for sl in (0, 1):
        wait_idx(sl)
        fire(sl, sl)

    def group(g, _):
        for sl in (0, 1, 2, 3):
            k = 4 * g + sl
            bs = sl % 2
            pltpu.make_async_copy(t_hbm.at[idxr[sl]], a[bs], sa[bs]).wait()
            pltpu.make_async_copy(t_hbm.at[idxc[sl]], b[bs], sb[bs]).wait()

            def relu_row(e, _):
                for j in range(H // 16):
                    slc = pl.ds(j * 16, 16)
                    a[bs][e, slc] = jnp.maximum(a[bs][e, slc] - b[bs][e, slc], 0.0)
                return 0
            lax.fori_loop(0, _KS, relu_row, 0)
            pltpu.sync_copy(a[bs], acc.at[idxc[sl]], add=True)

            @pl.when(k + 4 < _NCHS)
            def _():
                load_idx(k + 4, sl)

            @pl.when(k + 2 < _NCHS)
            def _():
                wait_idx((sl + 2) % 4)
                fire((sl + 2) % 4, bs)
        return 0
    lax.fori_loop(0, _NCHS // 4, group, 0)

    plsc.subcore_barrier()
    pltpu.sync_copy(acc.at[pl.ds(s * _RPT, _RPT)],
                    out_hbm.at[c, pl.ds(s * _RPT, _RPT)])


@functools.partial(
    pl.kernel,
    mesh=_mesh,
    out_type=jax.ShapeDtypeStruct((2, _NP, H), jnp.float32),
    scratch_types=[
        pltpu.VMEM((_KC,), jnp.int32),
        pltpu.VMEM((_KC,), jnp.int32),
        pltpu.VMEM((_KC,), jnp.int32),
        pltpu.VMEM((_KC,), jnp.int32),
        pltpu.VMEM((_KC,), jnp.int32),
        pltpu.VMEM((_KC,), jnp.int32),
        pltpu.VMEM((_KC,), jnp.int32),
        pltpu.VMEM((_KC,), jnp.int32),
        pltpu.VMEM((_KC, H), jnp.float32),
        pltpu.VMEM((_KC, H), jnp.float32),
        pltpu.VMEM_SHARED((_NP, H), jnp.float32),
        pltpu.SemaphoreType.DMA,
        pltpu.SemaphoreType.DMA,
        pltpu.SemaphoreType.DMA,
        pltpu.SemaphoreType.DMA,
        pltpu.SemaphoreType.DMA,
        pltpu.SemaphoreType.DMA,
    ],
)
def _sc_edge_c(t_hbm, row_hbm, col_hbm, out_hbm,
               idxr0, idxr1, idxr2, idxr3, idxc0, idxc1, idxc2, idxc3,
               a0, a1, acc, sa0, sa1, si0, si1, si2, si3):
    c = lax.axis_index("c")
    s = lax.axis_index("s")
    wid = c * _NTILE + s
    idxr = (idxr0, idxr1, idxr2, idxr3)
    idxc = (idxc0, idxc1, idxc2, idxc3)
    a = (a0, a1)
    sa = (sa0, sa1)
    si = (si0, si1, si2, si3)

    def zrow(r, _):
        for j in range(H // 16):
            a0[r, pl.ds(j * 16, 16)] = jnp.zeros((16,), jnp.float32)
        return 0
    lax.fori_loop(0, _KC, zrow, 0)
    for t in range(_RPT // _KC):
        pltpu.sync_copy(a0, acc.at[pl.ds(s * _RPT + t * _KC, _KC)])
    plsc.subcore_barrier()

    base = wid * _EPW

    def load_idx(k, sl):
        off = base + k * _KC
        pltpu.async_copy(row_hbm.at[pl.ds(off, _KC)], idxr[sl], si[sl])
        pltpu.async_copy(col_hbm.at[pl.ds(off, _KC)], idxc[sl], si[sl])

    def wait_idx(sl):
        pltpu.make_async_copy(row_hbm.at[pl.ds(0, _KC)], idxr[sl], si[sl]).wait()
        pltpu.make_async_copy(col_hbm.at[pl.ds(0, _KC)], idxc[sl], si[sl]).wait()

    def fire(sl, bs):
        pltpu.async_copy(t_hbm.at[idxr[sl]], a[bs], sa[bs])

    for sl in (0, 1, 2, 3):
        load_idx(sl, sl)
    for sl in (0, 1):
        wait_idx(sl)
        fire(sl, sl)

    def group(g, _):
        for sl in (0, 1, 2, 3):
            k = 4 * g + sl
            bs = sl % 2
            pltpu.make_async_copy(t_hbm.at[idxr[sl]], a[bs], sa[bs]).wait()
            pltpu.sync_copy(a[bs], acc.at[idxc[sl]], add=True)

            @pl.when(k + 4 < _NCHC)
            def _():
                load_idx(k + 4, sl)

            @pl.when(k + 2 < _NCHC)
            def _():
                wait_idx((sl + 2) % 4)
                fire((sl + 2) % 4, bs)
        return 0
    lax.fori_loop(0, _NCHC // 4, group, 0)

    plsc.subcore_barrier()
    pltpu.sync_copy(acc.at[pl.ds(s * _RPT, _RPT)],
                    out_hbm.at[c, pl.ds(s * _RPT, _RPT)])


_BR = 2000
_GRID = N // _BR


def _full(shape):
    return pl.BlockSpec(shape, lambda i: tuple(0 for _ in shape))


def _post_block(x, sp, cp, xwcs, v, dinv, lam):
    ssum = sp[0] + sp[1]
    csum = cp[0] + cp[1]
    t = jnp.maximum(dinv * (csum + xwcs) + v, 0.0)
    return t + lam * x * ssum


def _pre_block(x, wmc, wrt, bv, dinv):
    z = jnp.dot(x, wmc, preferred_element_type=jnp.float32)
    xwc = z[:, H:]
    v = bv - jnp.dot(xwc, wrt, preferred_element_type=jnp.float32)
    return z[:, :H], xwc * dinv, v


def _encpre_body(x_ref, wet_ref, be_ref, wmc_ref, wrt_ref, bv_ref, dinv_ref,
                 x_out, xwm_ref, xwcs_ref, v_ref):
    xx = jnp.maximum(
        jnp.dot(x_ref[...], wet_ref[...], preferred_element_type=jnp.float32)
        + be_ref[...], 0.0)
    x_out[...] = xx
    xwm, xwcs, v = _pre_block(xx, wmc_ref[...], wrt_ref[...], bv_ref[...],
                              dinv_ref[...])
    xwm_ref[...] = xwm
    xwcs_ref[...] = xwcs
    v_ref[...] = v


def _encpre(x, wet, be, wmc, wrt, bv, dinv):
    return pl.pallas_call(
        _encpre_body,
        grid=(_GRID,),
        in_specs=[pl.BlockSpec((_BR, H), lambda i: (i, 0)),
                  _full((H, H)), _full((1, H)),
                  _full((H, 2 * H)), _full((H, H)), _full((1, H)),
                  pl.BlockSpec((_BR, 1), lambda i: (i, 0))],
        out_specs=[pl.BlockSpec((_BR, H), lambda i: (i, 0))] * 4,
        out_shape=[jax.ShapeDtypeStruct((N, H), jnp.float32)] * 4,
    )(x, wet, be, wmc, wrt, bv, dinv)


def _postpre_body(x_ref, sp_ref, cp_ref, xwcs_ref, v_ref, dinv_ref, lam_ref,
                  wmc_ref, wrt_ref, bv_ref,
                  x_out, xwm_ref, xwcs_out, v_out):
    xn = _post_block(x_ref[...], sp_ref, cp_ref, xwcs_ref[...], v_ref[...],
                     dinv_ref[...], lam_ref[0, 0])
    x_out[...] = xn
    xwm, xwcs, v = _pre_block(xn, wmc_ref[...], wrt_ref[...], bv_ref[...],
                              dinv_ref[...])
    xwm_ref[...] = xwm
    xwcs_out[...] = xwcs
    v_out[...] = v


def _postpre(x, sp, cp, xwcs, v, dinv, lam, wmc, wrt, bv):
    return pl.pallas_call(
        _postpre_body,
        grid=(_GRID,),
        in_specs=[pl.BlockSpec((_BR, H), lambda i: (i, 0)),
                  pl.BlockSpec((2, _BR, H), lambda i: (0, i, 0)),
                  pl.BlockSpec((2, _BR, H), lambda i: (0, i, 0)),
                  pl.BlockSpec((_BR, H), lambda i: (i, 0)),
                  pl.BlockSpec((_BR, H), lambda i: (i, 0)),
                  pl.BlockSpec((_BR, 1), lambda i: (i, 0)),
                  _full((1, 1)),
                  _full((H, 2 * H)), _full((H, H)), _full((1, H))],
        out_specs=[pl.BlockSpec((_BR, H), lambda i: (i, 0))] * 4,
        out_shape=[jax.ShapeDtypeStruct((N, H), jnp.float32)] * 4,
    )(x, sp, cp, xwcs, v, dinv, lam, wmc, wrt, bv)


def _postdec_body(x_ref, sp_ref, cp_ref, xwcs_ref, v_ref, dinv_ref, lam_ref,
                  wdt_ref, bd_ref, o_ref):
    xn = _post_block(x_ref[...], sp_ref, cp_ref, xwcs_ref[...], v_ref[...],
                     dinv_ref[...], lam_ref[0, 0])
    o_ref[...] = jnp.dot(
        xn, wdt_ref[...], preferred_element_type=jnp.float32) + bd_ref[...]


def _postdec(x, sp, cp, xwcs, v, dinv, lam, wdt, bd):
    return pl.pallas_call(
        _postdec_body,
        grid=(_GRID,),
        in_specs=[pl.BlockSpec((_BR, H), lambda i: (i, 0)),
                  pl.BlockSpec((2, _BR, H), lambda i: (0, i, 0)),
                  pl.BlockSpec((2, _BR, H), lambda i: (0, i, 0)),
                  pl.BlockSpec((_BR, H), lambda i: (i, 0)),
                  pl.BlockSpec((_BR, H), lambda i: (i, 0)),
                  pl.BlockSpec((_BR, 1), lambda i: (i, 0)),
                  _full((1, 1)),
                  _full((H, NCLASS)), _full((1, NCLASS))],
        out_specs=pl.BlockSpec((_BR, NCLASS), lambda i: (i, 0)),
        out_shape=jax.ShapeDtypeStruct((N, NCLASS), jnp.float32),
    )(x, sp, cp, xwcs, v, dinv, lam, wdt, bd)


def kernel(x, edge_index, W_enc, b_enc, W_conv, b_conv, W_res, b_res,
           W_dec, b_dec, weight_mlp, lamda1):
    row = edge_index[0]
    col = edge_index[1]
    npad = _EP - E
    pad_r = jnp.arange(npad, dtype=jnp.int32) % N
    pad_c = (jnp.arange(npad, dtype=jnp.int32) % (_NP - N)) + N
    rowp = jnp.concatenate([row, pad_r])
    colp = jnp.concatenate([col, pad_c])

    degp = _sc_degree(colp)
    deg = degp[0, :N] + degp[1, :N] + 1.0
    dinv = jax.lax.rsqrt(deg).reshape(N, 1)

    wmc = jnp.concatenate([weight_mlp, W_conv.T], axis=1)
    bv = (b_conv - b_res).reshape(1, H)
    lam = lamda1.reshape(1, 1)

    X, xwm, xwcs, v = _encpre(x, W_enc.T, b_enc.reshape(1, H),
                              wmc, W_res.T, bv, dinv)
    sp = _sc_edge_s(xwm, rowp, colp)
    cp = _sc_edge_c(xwcs, rowp, colp)
    X, xwm, xwcs, v = _postpre(X, sp, cp, xwcs, v, dinv, lam,
                               wmc, W_res.T, bv)
    sp = _sc_edge_s(xwm, rowp, colp)
    cp = _sc_edge_c(xwcs, rowp, colp)
    return _postdec(X, sp, cp, xwcs, v, dinv, lam,
                    W_dec.T, b_dec.reshape(1, NCLASS))


# speedup vs baseline: 18.4772x; 1.0398x over previous
import functools

import jax
import jax.numpy as jnp
from jax import lax
from jax.experimental import pallas as pl
from jax.experimental.pallas import tpu as pltpu
from jax.experimental.pallas import tpu_sc as plsc

N = 10000
E = 320000
H = 128
NCLASS = 40

_NTILE = 16
_NW = 32
_NP = 10240
_EP = _NW * _NP
_EPW = _EP // _NW
_RPT = _NP // _NTILE

_KS = 80
_NCHS = _EPW // _KS
_KC = 160
_NCHC = _EPW // _KC
_KD = 640
_NCHD = _EPW // _KD

_mesh = plsc.VectorSubcoreMesh(core_axis_name="c", subcore_axis_name="s")


@functools.partial(
    pl.kernel,
    mesh=_mesh,
    out_type=jax.ShapeDtypeStruct((2, _NP), jnp.float32),
    scratch_types=[
        pltpu.VMEM((_KD,), jnp.int32),
        pltpu.VMEM((_KD,), jnp.float32),
        pltpu.VMEM((_RPT,), jnp.float32),
        pltpu.VMEM_SHARED((_NP,), jnp.float32),
    ],
)
def _sc_degree(col_hbm, out_hbm, idxc, ones, zbuf, acc):
    c = lax.axis_index("c")
    s = lax.axis_index("s")
    wid = c * _NTILE + s

    def fill(i, _):
        ones[pl.ds(i * 16, 16)] = jnp.full((16,), 1.0, jnp.float32)
        zbuf[pl.ds(i * 16, 16)] = jnp.zeros((16,), jnp.float32)
        return 0
    lax.fori_loop(0, _RPT // 16, fill, 0)

    pltpu.sync_copy(zbuf, acc.at[pl.ds(s * _RPT, _RPT)])
    plsc.subcore_barrier()

    base = wid * _EPW

    def chunk(k, _):
        pltpu.sync_copy(col_hbm.at[pl.ds(base + k * _KD, _KD)], idxc)
        pltpu.sync_copy(ones, acc.at[idxc], add=True)
        return 0
    lax.fori_loop(0, _NCHD, chunk, 0)

    plsc.subcore_barrier()
    pltpu.sync_copy(acc.at[pl.ds(s * _RPT, _RPT)],
                    out_hbm.at[c, pl.ds(s * _RPT, _RPT)])


@functools.partial(
    pl.kernel,
    mesh=_mesh,
    out_type=[jax.ShapeDtypeStruct((2, _NP, H), jnp.float32),
              jax.ShapeDtypeStruct((2, _NP, H), jnp.float32)],
    scratch_types=[
        pltpu.VMEM((_KS,), jnp.int32),
        pltpu.VMEM((_KS,), jnp.int32),
        pltpu.VMEM((_KS,), jnp.int32),
        pltpu.VMEM((_KS,), jnp.int32),
        pltpu.VMEM((_KS,), jnp.int32),
        pltpu.VMEM((_KS,), jnp.int32),
        pltpu.VMEM((_KS,), jnp.int32),
        pltpu.VMEM((_KS,), jnp.int32),
        pltpu.VMEM((_KS, H), jnp.float32),
        pltpu.VMEM((_KS, H), jnp.float32),
        pltpu.VMEM((_KS, H), jnp.float32),
        pltpu.VMEM((_KS, H), jnp.float32),
        pltpu.VMEM_SHARED((_NP, H), jnp.float32),
        pltpu.SemaphoreType.DMA,
        pltpu.SemaphoreType.DMA,
        pltpu.SemaphoreType.DMA,
        pltpu.SemaphoreType.DMA,
        pltpu.SemaphoreType.DMA,
        pltpu.SemaphoreType.DMA,
        pltpu.SemaphoreType.DMA,
        pltpu.SemaphoreType.DMA,
    ],
)
def _sc_edges(tm_hbm, tc_hbm, row_hbm, col_hbm, outs_hbm, outc_hbm,
              idxr0, idxr1, idxr2, idxr3, idxc0, idxc1, idxc2, idxc3,
              a0, a1, b0, b1, acc,
              g0, g1, g2, g3, si0, si1, si2, si3):
    c = lax.axis_index("c")
    s = lax.axis_index("s")
    wid = c * _NTILE + s
    idxr = (idxr0, idxr1, idxr2, idxr3)
    idxc = (idxc0, idxc1, idxc2, idxc3)
    buf = (a0, a1, b0, b1)
    g = (g0, g1, g2, g3)
    si = (si0, si1, si2, si3)
    base = wid * _EPW

    def zero_own_slice(stage):
        def zrow(r, _):
            for j in range(H // 16):
                stage[r, pl.ds(j * 16, 16)] = jnp.zeros((16,), jnp.float32)
            return 0
        lax.fori_loop(0, _KS, zrow, 0)
        for t in range(_RPT // _KS):
            pltpu.sync_copy(stage, acc.at[pl.ds(s * _RPT + t * _KS, _KS)])

    def load_idx(k, sl):
        off = base + k * _KS
        pltpu.async_copy(row_hbm.at[pl.ds(off, _KS)], idxr[sl], si[sl])
        pltpu.async_copy(col_hbm.at[pl.ds(off, _KS)], idxc[sl], si[sl])

    def wait_idx(sl):
        pltpu.make_async_copy(row_hbm.at[pl.ds(0, _KS)], idxr[sl], si[sl]).wait()
        pltpu.make_async_copy(col_hbm.at[pl.ds(0, _KS)], idxc[sl], si[sl]).wait()

    def writeout(out_hbm):
        pltpu.sync_copy(acc.at[pl.ds(s * _RPT, _RPT)],
                        out_hbm.at[c, pl.ds(s * _RPT, _RPT)])

    zero_own_slice(b0)
    plsc.subcore_barrier()

    def fire_s(sl, bs):
        pltpu.async_copy(tm_hbm.at[idxr[sl]], buf[bs], g[bs])
        pltpu.async_copy(tm_hbm.at[idxc[sl]], buf[2 + bs], g[2 + bs])

    for sl in (0, 1, 2, 3):
        load_idx(sl, sl)
    for sl in (0, 1):
        wait_idx(sl)
        fire_s(sl, sl)

    def group_s(gi, _):
        for sl in (0, 1, 2, 3):
            k = 4 * gi + sl
            bs = sl % 2
            pltpu.make_async_copy(tm_hbm.at[idxr[sl]], buf[bs], g[bs]).wait()
            pltpu.make_async_copy(tm_hbm.at[idxc[sl]], buf[2 + bs], g[2 + bs]).wait()

            def relu_row(e, _):
                for j in range(H // 16):
                    slc = pl.ds(j * 16, 16)
                    buf[bs][e, slc] = jnp.maximum(
                        buf[bs][e, slc] - buf[2 + bs][e, slc], 0.0)
                return 0
            lax.fori_loop(0, _KS, relu_row, 0)
            pltpu.sync_copy(buf[bs], acc.at[idxc[sl]], add=True)

            @pl.when(k + 4 < _NCHS)
            def _():
                load_idx(k + 4, sl)

            @pl.when(k + 2 < _NCHS)
            def _():
                wait_idx((sl + 2) % 4)
                fire_s((sl + 2) % 4, bs)
        return 0
    lax.fori_loop(0, _NCHS // 4, group_s, 0)

    plsc.subcore_barrier()
    writeout(outs_hbm)
    zero_own_slice(b0)
    plsc.subcore_barrier()

    def fire_c(sl):
        pltpu.async_copy(tc_hbm.at[idxr[sl]], buf[sl], g[sl])

    for sl in (0, 1, 2, 3):
        load_idx(sl, sl)
    for sl in (0, 1, 2):
        wait_idx(sl)
        fire_c(sl)

    def group_c(gi, _):
        for sl in (0, 1, 2, 3):
            k = 4 * gi + sl
            pltpu.make_async_copy(tc_hbm.at[idxr[sl]], buf[sl], g[sl]).wait()
            pltpu.sync_copy(buf[sl], acc.at[idxc[sl]], add=True)

            @pl.when(k + 4 < _NCHS)
            def _():
                load_idx(k + 4, sl)

            @pl.when(k + 3 < _NCHS)
            def _():
                wait_idx((sl + 3) % 4)
                fire_c((sl + 3) % 4)
        return 0
    lax.fori_loop(0, _NCHS // 4, group_c, 0)

    plsc.subcore_barrier()
    writeout(outc_hbm)


_BR = 2000
_GRID = N // _BR


def _full(shape):
    return pl.BlockSpec(shape, lambda i: tuple(0 for _ in shape))


def _post_block(x, sp, cp, xwcs, v, dinv, lam):
    ssum = sp[0] + sp[1]
    csum = cp[0] + cp[1]
    t = jnp.maximum(dinv * (csum + xwcs) + v, 0.0)
    return t + lam * x * ssum


def _pre_block(x, wmc, wrt, bv, dinv):
    z = jnp.dot(x, wmc, preferred_element_type=jnp.float32)
    xwc = z[:, H:]
    v = bv - jnp.dot(xwc, wrt, preferred_element_type=jnp.float32)
    return z[:, :H], xwc * dinv, v


def _encpre_body(x_ref, wet_ref, be_ref, wmc_ref, wrt_ref, bv_ref, dinv_ref,
                 x_out, xwm_ref, xwcs_ref, v_ref):
    xx = jnp.maximum(
        jnp.dot(x_ref[...], wet_ref[...], preferred_element_type=jnp.float32)
        + be_ref[...], 0.0)
    x_out[...] = xx
    xwm, xwcs, v = _pre_block(xx, wmc_ref[...], wrt_ref[...], bv_ref[...],
                              dinv_ref[...])
    xwm_ref[...] = xwm
    xwcs_ref[...] = xwcs
    v_ref[...] = v


def _encpre(x, wet, be, wmc, wrt, bv, dinv):
    return pl.pallas_call(
        _encpre_body,
        grid=(_GRID,),
        in_specs=[pl.BlockSpec((_BR, H), lambda i: (i, 0)),
                  _full((H, H)), _full((1, H)),
                  _full((H, 2 * H)), _full((H, H)), _full((1, H)),
                  pl.BlockSpec((_BR, 1), lambda i: (i, 0))],
        out_specs=[pl.BlockSpec((_BR, H), lambda i: (i, 0))] * 4,
        out_shape=[jax.ShapeDtypeStruct((N, H), jnp.float32)] * 4,
    )(x, wet, be, wmc, wrt, bv, dinv)


def _postpre_body(x_ref, sp_ref, cp_ref, xwcs_ref, v_ref, dinv_ref, lam_ref,
                  wmc_ref, wrt_ref, bv_ref,
                  x_out, xwm_ref, xwcs_out, v_out):
    xn = _post_block(x_ref[...], sp_ref, cp_ref, xwcs_ref[...], v_ref[...],
                     dinv_ref[...], lam_ref[0, 0])
    x_out[...] = xn
    xwm, xwcs, v = _pre_block(xn, wmc_ref[...], wrt_ref[...], bv_ref[...],
                              dinv_ref[...])
    xwm_ref[...] = xwm
    xwcs_out[...] = xwcs
    v_out[...] = v


def _postpre(x, sp, cp, xwcs, v, dinv, lam, wmc, wrt, bv):
    return pl.pallas_call(
        _postpre_body,
        grid=(_GRID,),
        in_specs=[pl.BlockSpec((_BR, H), lambda i: (i, 0)),
                  pl.BlockSpec((2, _BR, H), lambda i: (0, i, 0)),
                  pl.BlockSpec((2, _BR, H), lambda i: (0, i, 0)),
                  pl.BlockSpec((_BR, H), lambda i: (i, 0)),
                  pl.BlockSpec((_BR, H), lambda i: (i, 0)),
                  pl.BlockSpec((_BR, 1), lambda i: (i, 0)),
                  _full((1, 1)),
                  _full((H, 2 * H)), _full((H, H)), _full((1, H))],
        out_specs=[pl.BlockSpec((_BR, H), lambda i: (i, 0))] * 4,
        out_shape=[jax.ShapeDtypeStruct((N, H), jnp.float32)] * 4,
    )(x, sp, cp, xwcs, v, dinv, lam, wmc, wrt, bv)


def _postdec_body(x_ref, sp_ref, cp_ref, xwcs_ref, v_ref, dinv_ref, lam_ref,
                  wdt_ref, bd_ref, o_ref):
    xn = _post_block(x_ref[...], sp_ref, cp_ref, xwcs_ref[...], v_ref[...],
                     dinv_ref[...], lam_ref[0, 0])
    o_ref[...] = jnp.dot(
        xn, wdt_ref[...], preferred_element_type=jnp.float32) + bd_ref[...]


def _postdec(x, sp, cp, xwcs, v, dinv, lam, wdt, bd):
    return pl.pallas_call(
        _postdec_body,
        grid=(_GRID,),
        in_specs=[pl.BlockSpec((_BR, H), lambda i: (i, 0)),
                  pl.BlockSpec((2, _BR, H), lambda i: (0, i, 0)),
                  pl.BlockSpec((2, _BR, H), lambda i: (0, i, 0)),
                  pl.BlockSpec((_BR, H), lambda i: (i, 0)),
                  pl.BlockSpec((_BR, H), lambda i: (i, 0)),
                  pl.BlockSpec((_BR, 1), lambda i: (i, 0)),
                  _full((1, 1)),
                  _full((H, NCLASS)), _full((1, NCLASS))],
        out_specs=pl.BlockSpec((_BR, NCLASS), lambda i: (i, 0)),
        out_shape=jax.ShapeDtypeStruct((N, NCLASS), jnp.float32),
    )(x, sp, cp, xwcs, v, dinv, lam, wdt, bd)


def kernel(x, edge_index, W_enc, b_enc, W_conv, b_conv, W_res, b_res,
           W_dec, b_dec, weight_mlp, lamda1):
    row = edge_index[0]
    col = edge_index[1]
    npad = _EP - E
    pad_r = jnp.arange(npad, dtype=jnp.int32) % N
    pad_c = (jnp.arange(npad, dtype=jnp.int32) % (_NP - N)) + N
    rowp = jnp.concatenate([row, pad_r])
    colp = jnp.concatenate([col, pad_c])

    degp = _sc_degree(colp)
    deg = degp[0, :N] + degp[1, :N] + 1.0
    dinv = jax.lax.rsqrt(deg).reshape(N, 1)

    wmc = jnp.concatenate([weight_mlp, W_conv.T], axis=1)
    bv = (b_conv - b_res).reshape(1, H)
    lam = lamda1.reshape(1, 1)

    X, xwm, xwcs, v = _encpre(x, W_enc.T, b_enc.reshape(1, H),
                              wmc, W_res.T, bv, dinv)
    sp, cp = _sc_edges(xwm, xwcs, rowp, colp)
    X, xwm, xwcs, v = _postpre(X, sp, cp, xwcs, v, dinv, lam,
                               wmc, W_res.T, bv)
    sp, cp = _sc_edges(xwm, xwcs, rowp, colp)
    return _postdec(X, sp, cp, xwcs, v, dinv, lam,
                    W_dec.T, b_dec.reshape(1, NCLASS))
